# baseline (speedup 1.0000x reference)
import jax
import jax.numpy as jnp
from jax.experimental import pallas as pl
from jax.experimental.pallas import tpu as pltpu

N = 8192
BLK = 512


def _matvec_body(f_ref, w_ref, out_ref):
    out_ref[...] = jax.lax.dot_general(
        w_ref[...], f_ref[...],
        dimension_numbers=(((1,), (0,)), ((), ())),
        preferred_element_type=jnp.float32,
    )


def _threshold_body(x_ref, nx_ref, nf_ref):
    v = x_ref[...]
    theta = jnp.mean(v)
    nx_ref[...] = v
    nf_ref[...] = (v > theta).astype(jnp.float32)


def kernel(x, is_firing, W):
    f2 = is_firing.reshape(N, 1)
    node_x2 = pl.pallas_call(
        _matvec_body,
        grid=(N // BLK,),
        in_specs=[
            pl.BlockSpec((N, 1), lambda i: (0, 0)),
            pl.BlockSpec((BLK, N), lambda i: (i, 0)),
        ],
        out_specs=pl.BlockSpec((BLK, 1), lambda i: (i, 0)),
        out_shape=jax.ShapeDtypeStruct((N, 1), jnp.float32),
    )(f2, W)

    node_x8 = node_x2.reshape(8, N // 8)
    nx, nf = pl.pallas_call(
        _threshold_body,
        out_shape=(
            jax.ShapeDtypeStruct((8, N // 8), jnp.float32),
            jax.ShapeDtypeStruct((8, N // 8), jnp.float32),
        ),
    )(node_x8)
    return nx.reshape(N), nf.reshape(N)
